# baseline (speedup 1.0000x reference)
import functools

import jax
import jax.numpy as jnp
from jax import lax
from jax.experimental import pallas as pl
from jax.experimental.pallas import tpu as pltpu


def _fused_body(xb_ref, sel_ref, w1_ref, w2_ref, g1_ref, b1_ref, g2_ref,
                b2_ref, o_ref, *, eps):
    N, C, Hp, W4 = xb_ref.shape
    Wp = W4 // 4
    HW = Hp * Wp
    M = N * HW
    inv_count = 1.0 / float(M)

    xv = xb_ref[...]
    vert = jnp.maximum(xv[..., 0:2 * Wp], xv[..., 2 * Wp:4 * Wp])
    vd = vert.astype(jnp.bfloat16).reshape(N * C * Hp, 2 * Wp)
    pc = jnp.dot(vd, sel_ref[...],
                 preferred_element_type=jnp.float32)
    pooled = jnp.maximum(pc[:, 0:Wp], pc[:, Wp:2 * Wp])
    pooled = pooled.astype(jnp.bfloat16).reshape(N, C, Hp, Wp)
    xp = jnp.transpose(pooled, (0, 2, 3, 1))

    def conv3x3(a, w_ref):
        apad = jnp.pad(a, ((0, 0), (1, 1), (1, 1), (0, 0)))
        b = jnp.concatenate([apad[:, :, dx:dx + Wp, :] for dx in range(3)],
                            axis=3)
        z = jnp.dot(b.reshape(N * (Hp + 2) * Wp, 3 * C), w_ref[...],
                    preferred_element_type=jnp.float32)
        z = z.reshape(N, Hp + 2, Wp, 3 * C)
        return (z[:, 0:Hp, :, 0:C] + z[:, 1:Hp + 1, :, C:2 * C]
                + z[:, 2:Hp + 2, :, 2 * C:3 * C])

    def bn_relu(y, g_ref, b_ref):
        s = jnp.sum(y, axis=(0, 1, 2), keepdims=True)
        ss = jnp.sum(y * y, axis=(0, 1, 2), keepdims=True)
        mean = s * inv_count
        var = jnp.maximum(ss * inv_count - mean * mean, 0.0)
        scale = g_ref[...].reshape(1, 1, 1, C) * lax.rsqrt(var + eps)
        shift = b_ref[...].reshape(1, 1, 1, C) - mean * scale
        return jnp.maximum(y * scale + shift, 0.0)

    a1 = bn_relu(conv3x3(xp, w1_ref), g1_ref, b1_ref).astype(jnp.bfloat16)
    a2 = bn_relu(conv3x3(a1, w2_ref), g2_ref, b2_ref)
    o_ref[...] = jnp.transpose(a2.reshape(N, HW, C), (0, 2, 1))


@jax.jit
def kernel(x, conv1_w, bn1_g, bn1_b, conv2_w, bn2_g, bn2_b):
    eps = 1e-5
    N, C, H, W = x.shape
    Hp, Wp = H // 2, W // 2
    Cout = conv1_w.shape[3]
    xb = x.reshape(N, C, Hp, 2 * W)
    lane = jnp.arange(2 * Wp)
    sel = ((lane[:, None] == 2 * (lane[None, :] % Wp) + lane[None, :] // Wp)
           .astype(jnp.bfloat16))
    w1c = conv1_w.transpose(1, 2, 0, 3).reshape(3 * C, 3 * Cout)
    w2c = conv2_w.transpose(1, 2, 0, 3).reshape(3 * Cout, 3 * Cout)
    body = functools.partial(_fused_body, eps=eps)
    out = pl.pallas_call(
        body,
        out_shape=jax.ShapeDtypeStruct((N, Cout, Hp * Wp), jnp.float32),
        grid=(1,),
        in_specs=[
            pl.BlockSpec((N, C, Hp, 2 * W), lambda i: (0, 0, 0, 0)),
            pl.BlockSpec((2 * Wp, 2 * Wp), lambda i: (0, 0)),
            pl.BlockSpec((3 * C, 3 * Cout), lambda i: (0, 0)),
            pl.BlockSpec((3 * Cout, 3 * Cout), lambda i: (0, 0)),
            pl.BlockSpec((1, Cout), lambda i: (0, 0)),
            pl.BlockSpec((1, Cout), lambda i: (0, 0)),
            pl.BlockSpec((1, Cout), lambda i: (0, 0)),
            pl.BlockSpec((1, Cout), lambda i: (0, 0)),
        ],
        out_specs=pl.BlockSpec((N, Cout, Hp * Wp), lambda i: (0, 0, 0)),
        compiler_params=pltpu.CompilerParams(
            dimension_semantics=("arbitrary",)),
    )(xb, sel, w1c.astype(jnp.bfloat16), w2c.astype(jnp.bfloat16),
      bn1_g.reshape(1, Cout), bn1_b.reshape(1, Cout),
      bn2_g.reshape(1, Cout), bn2_b.reshape(1, Cout))
    return out.reshape(N, Cout, Hp, Wp)


# speedup vs baseline: 1.0248x; 1.0248x over previous
import functools

import jax
import jax.numpy as jnp
from jax import lax
from jax.experimental import pallas as pl
from jax.experimental.pallas import tpu as pltpu


def _fused_body(xb_ref, sel_ref, w1_ref, w2_ref, g1_ref, b1_ref, g2_ref,
                b2_ref, o_ref, *, eps):
    N, C, Hp, W4 = xb_ref.shape
    Wp = W4 // 4
    HW = Hp * Wp
    M = N * HW
    inv_count = 1.0 / float(M)

    xv = xb_ref[...]
    vert = jnp.maximum(xv[..., 0:2 * Wp], xv[..., 2 * Wp:4 * Wp])
    vd = vert.astype(jnp.bfloat16).reshape(N * C * Hp, 2 * Wp)
    pc = jnp.dot(vd, sel_ref[...],
                 preferred_element_type=jnp.float32)
    pooled = jnp.maximum(pc[:, 0:Wp], pc[:, Wp:2 * Wp])
    pooled = pooled.astype(jnp.bfloat16).reshape(N, C, Hp, Wp)
    xp = jnp.transpose(jnp.transpose(pooled, (0, 2, 1, 3)), (0, 1, 3, 2))

    def conv3x3(a, w_ref):
        apad = jnp.pad(a, ((0, 0), (1, 1), (1, 1), (0, 0)))
        b = jnp.concatenate([apad[:, :, dx:dx + Wp, :] for dx in range(3)],
                            axis=3)
        z = jnp.dot(b.reshape(N * (Hp + 2) * Wp, 3 * C), w_ref[...],
                    preferred_element_type=jnp.float32)
        z = z.reshape(N, Hp + 2, Wp, 3 * C)
        return (z[:, 0:Hp, :, 0:C] + z[:, 1:Hp + 1, :, C:2 * C]
                + z[:, 2:Hp + 2, :, 2 * C:3 * C])

    def bn_relu(y, g_ref, b_ref):
        s = jnp.sum(y, axis=(0, 1, 2), keepdims=True)
        ss = jnp.sum(y * y, axis=(0, 1, 2), keepdims=True)
        mean = s * inv_count
        var = jnp.maximum(ss * inv_count - mean * mean, 0.0)
        scale = g_ref[...].reshape(1, 1, 1, C) * lax.rsqrt(var + eps)
        shift = b_ref[...].reshape(1, 1, 1, C) - mean * scale
        return jnp.maximum(y * scale + shift, 0.0)

    a1 = bn_relu(conv3x3(xp, w1_ref), g1_ref, b1_ref).astype(jnp.bfloat16)
    a2 = bn_relu(conv3x3(a1, w2_ref), g2_ref, b2_ref)
    o_ref[...] = jnp.transpose(a2.reshape(N, HW, C), (0, 2, 1))


@jax.jit
def kernel(x, conv1_w, bn1_g, bn1_b, conv2_w, bn2_g, bn2_b):
    eps = 1e-5
    N, C, H, W = x.shape
    Hp, Wp = H // 2, W // 2
    Cout = conv1_w.shape[3]
    xb = x.reshape(N, C, Hp, 2 * W)
    lane = jnp.arange(2 * Wp)
    sel = ((lane[:, None] == 2 * (lane[None, :] % Wp) + lane[None, :] // Wp)
           .astype(jnp.bfloat16))
    w1c = conv1_w.transpose(1, 2, 0, 3).reshape(3 * C, 3 * Cout)
    w2c = conv2_w.transpose(1, 2, 0, 3).reshape(3 * Cout, 3 * Cout)
    body = functools.partial(_fused_body, eps=eps)
    out = pl.pallas_call(
        body,
        out_shape=jax.ShapeDtypeStruct((N, Cout, Hp * Wp), jnp.float32),
        grid=(1,),
        in_specs=[
            pl.BlockSpec((N, C, Hp, 2 * W), lambda i: (0, 0, 0, 0)),
            pl.BlockSpec((2 * Wp, 2 * Wp), lambda i: (0, 0)),
            pl.BlockSpec((3 * C, 3 * Cout), lambda i: (0, 0)),
            pl.BlockSpec((3 * Cout, 3 * Cout), lambda i: (0, 0)),
            pl.BlockSpec((1, Cout), lambda i: (0, 0)),
            pl.BlockSpec((1, Cout), lambda i: (0, 0)),
            pl.BlockSpec((1, Cout), lambda i: (0, 0)),
            pl.BlockSpec((1, Cout), lambda i: (0, 0)),
        ],
        out_specs=pl.BlockSpec((N, Cout, Hp * Wp), lambda i: (0, 0, 0)),
        compiler_params=pltpu.CompilerParams(
            dimension_semantics=("arbitrary",)),
    )(xb, sel, w1c.astype(jnp.bfloat16), w2c.astype(jnp.bfloat16),
      bn1_g.reshape(1, Cout), bn1_b.reshape(1, Cout),
      bn2_g.reshape(1, Cout), bn2_b.reshape(1, Cout))
    return out.reshape(N, Cout, Hp, Wp)
